# Initial kernel scaffold; baseline (speedup 1.0000x reference)
import jax
import jax.numpy as jnp
from jax.experimental import pallas as pl


def kernel(x_m, x_d, mm_edge_index, dd_edge_index, gx1_W, gx1_as, gx1_ad, gx1_b, gx2_W, gx2_as, gx2_ad, gx2_b, gy1_W, gy1_as, gy1_ad, gy1_b, gy2_W, gy2_as, gy2_ad, gy2_b, lx1_W, lx1_b, lx2_W, lx2_b, lx3_W, lx3_b, ly1_W, ly1_b, ly2_W, ly2_b, ly3_W, ly3_b):
    raise NotImplementedError("write your pallas kernel here")


# speedup vs baseline: 20.0769x; 20.0769x over previous
import functools
import jax
import jax.numpy as jnp
from jax import lax
from jax.experimental import pallas as pl
from jax.experimental.pallas import tpu as pltpu
from jax.experimental.pallas import tpu_sc as plsc

F32 = jnp.float32
I32 = jnp.int32

M = 10000
D = 2000
NC = 2
NS = 16
NW = NC * NS
CH = 80

_HIGH = jax.lax.Precision.HIGHEST


def _dot(a, b):
    return jnp.dot(a, b, precision=_HIGH, preferred_element_type=F32)


def _atab_heads(a_cols, d_cols, H):
    return [jnp.concatenate([a_cols[h], d_cols[h]], axis=1) for h in range(H)]


def _proj_raw(x, W, att_s, att_d, N, Fin, H, C, OW):
    F = H * C
    NB = F // OW
    NBLK = N // 200
    attS = att_s.reshape(NB, OW)
    attD = att_d.reshape(NB, OW)

    def body(x_ref, w_ref, as_ref, ad_ref, *out_refs):
        xw_refs = out_refs[:NB]
        atab_refs = out_refs[NB:NB + H]
        cs_ref, cd_ref = out_refs[NB + H], out_refs[NB + H + 1]
        nb = pl.program_id(0)
        xb = x_ref[...]
        a_cols = [None] * H
        d_cols = [None] * H
        for j in range(NB):
            h = (j * OW) // C
            xw = _dot(xb, w_ref[:, j * OW:(j + 1) * OW])
            xw_refs[j][...] = xw
            sa = jnp.sum(xw * as_ref[j:j + 1, :], axis=1, keepdims=True)
            da = jnp.sum(xw * ad_ref[j:j + 1, :], axis=1, keepdims=True)
            a_cols[h] = sa if a_cols[h] is None else a_cols[h] + sa
            d_cols[h] = da if d_cols[h] is None else d_cols[h] + da
        tabs = _atab_heads(a_cols, d_cols, H)
        for h in range(H):
            atab_refs[h][...] = tabs[h]
        ms = jnp.max(jnp.concatenate(a_cols, axis=1))
        md = jnp.max(jnp.concatenate(d_cols, axis=1))

        @pl.when(nb == 0)
        def _():
            cs_ref[...] = jnp.full((8, 128), -1e30, F32)
            cd_ref[...] = jnp.full((8, 128), -1e30, F32)

        cs_ref[...] = jnp.maximum(cs_ref[...], jnp.full((8, 128), ms, F32))
        cd_ref[...] = jnp.maximum(cd_ref[...], jnp.full((8, 128), md, F32))

    out_shapes = [jax.ShapeDtypeStruct((N, OW), F32) for _ in range(NB)]
    out_shapes += [jax.ShapeDtypeStruct((N, 2), F32) for _ in range(H)]
    out_shapes += [jax.ShapeDtypeStruct((8, 128), F32),
                   jax.ShapeDtypeStruct((8, 128), F32)]
    out_specs = [pl.BlockSpec((200, OW), lambda nb: (nb, 0)) for _ in range(NB)]
    out_specs += [pl.BlockSpec((200, 2), lambda nb: (nb, 0)) for _ in range(H)]
    out_specs += [pl.BlockSpec((8, 128), lambda nb: (0, 0)),
                  pl.BlockSpec((8, 128), lambda nb: (0, 0))]
    outs = pl.pallas_call(
        body,
        grid=(NBLK,),
        in_specs=[pl.BlockSpec((200, Fin), lambda nb: (nb, 0)),
                  pl.BlockSpec((Fin, F), lambda nb: (0, 0)),
                  pl.BlockSpec((NB, OW), lambda nb: (0, 0)),
                  pl.BlockSpec((NB, OW), lambda nb: (0, 0))],
        out_specs=out_specs,
        out_shape=out_shapes,
    )(x, W, attS, attD)
    xw_list = list(outs[:NB])
    atabs = [outs[NB + h] for h in range(H)]
    cs, cd = outs[NB + H], outs[NB + H + 1]
    c = jax.nn.leaky_relu(jnp.max(cs) + jnp.max(cd), 0.2)
    c16 = jnp.full((16,), c, F32)
    return xw_list, atabs, c16


def _proj_merge(p_list, dp, bias_prev, W, att_s, att_d, N, C_in, H, C,
                OW_in, OW):
    NBk = len(p_list)
    Fin = NBk * OW_in
    F = H * C
    NB = F // OW
    NBLK = N // 200
    attS = att_s.reshape(NB, OW)
    attD = att_d.reshape(NB, OW)
    bprev = bias_prev.reshape(NBk, OW_in)
    dp2 = dp.reshape(2, N, 8)

    def body(*refs):
        p_refs = refs[:NBk]
        dp_ref, b_ref, w_ref, as_ref, ad_ref = refs[NBk:NBk + 5]
        out_refs = refs[NBk + 5:]
        xw_refs = out_refs[:NB]
        atab_refs = out_refs[NB:NB + H]
        cs_ref, cd_ref = out_refs[NB + H], out_refs[NB + H + 1]
        nb = pl.program_id(0)
        dmerged = dp_ref[0] + dp_ref[1]
        accs = [None] * NB
        for k in range(NBk):
            hk = (k * OW_in) // C_in
            pm = p_refs[k][0] + p_refs[k][1]
            dinv = 1.0 / (dmerged[:, hk:hk + 1] + 1e-16)
            Xk = jnp.maximum(pm * dinv + b_ref[k:k + 1, :], 0.0)
            for j in range(NB):
                contrib = _dot(Xk, w_ref[k * OW_in:(k + 1) * OW_in,
                                         j * OW:(j + 1) * OW])
                accs[j] = contrib if accs[j] is None else accs[j] + contrib
        a_cols = [None] * H
        d_cols = [None] * H
        for j in range(NB):
            h = (j * OW) // C
            xw_refs[j][...] = accs[j]
            sa = jnp.sum(accs[j] * as_ref[j:j + 1, :], axis=1, keepdims=True)
            da = jnp.sum(accs[j] * ad_ref[j:j + 1, :], axis=1, keepdims=True)
            a_cols[h] = sa if a_cols[h] is None else a_cols[h] + sa
            d_cols[h] = da if d_cols[h] is None else d_cols[h] + da
        tabs = _atab_heads(a_cols, d_cols, H)
        for h in range(H):
            atab_refs[h][...] = tabs[h]
        ms = jnp.max(jnp.concatenate(a_cols, axis=1))
        md = jnp.max(jnp.concatenate(d_cols, axis=1))

        @pl.when(nb == 0)
        def _():
            cs_ref[...] = jnp.full((8, 128), -1e30, F32)
            cd_ref[...] = jnp.full((8, 128), -1e30, F32)

        cs_ref[...] = jnp.maximum(cs_ref[...], jnp.full((8, 128), ms, F32))
        cd_ref[...] = jnp.maximum(cd_ref[...], jnp.full((8, 128), md, F32))

    in_specs = [pl.BlockSpec((2, 200, OW_in), lambda nb: (0, nb, 0))
                for _ in range(NBk)]
    in_specs += [pl.BlockSpec((2, 200, 8), lambda nb: (0, nb, 0)),
                 pl.BlockSpec((NBk, OW_in), lambda nb: (0, 0)),
                 pl.BlockSpec((Fin, F), lambda nb: (0, 0)),
                 pl.BlockSpec((NB, OW), lambda nb: (0, 0)),
                 pl.BlockSpec((NB, OW), lambda nb: (0, 0))]
    out_shapes = [jax.ShapeDtypeStruct((N, OW), F32) for _ in range(NB)]
    out_shapes += [jax.ShapeDtypeStruct((N, 2), F32) for _ in range(H)]
    out_shapes += [jax.ShapeDtypeStruct((8, 128), F32),
                   jax.ShapeDtypeStruct((8, 128), F32)]
    out_specs = [pl.BlockSpec((200, OW), lambda nb: (nb, 0)) for _ in range(NB)]
    out_specs += [pl.BlockSpec((200, 2), lambda nb: (nb, 0)) for _ in range(H)]
    out_specs += [pl.BlockSpec((8, 128), lambda nb: (0, 0)),
                  pl.BlockSpec((8, 128), lambda nb: (0, 0))]
    outs = pl.pallas_call(
        body,
        grid=(NBLK,),
        in_specs=in_specs,
        out_specs=out_specs,
        out_shape=out_shapes,
    )(*p_list, dp2, bprev, W, attS, attD)
    xw_list = list(outs[:NB])
    atabs = [outs[NB + h] for h in range(H)]
    cs, cd = outs[NB + H], outs[NB + H + 1]
    c = jax.nn.leaky_relu(jnp.max(cs) + jnp.max(cd), 0.2)
    c16 = jnp.full((16,), c, F32)
    return xw_list, atabs, c16


def _merge_mlp(p_list, dp, bias_prev, W1, b1, W2, b2, W3, b3, N, OW_in):
    NBk = len(p_list)
    NBLK = N // 200
    dp2 = dp.reshape(2, N, 8)
    bprev = bias_prev.reshape(NBk, OW_in)

    def body(*refs):
        p_refs = refs[:NBk]
        (dp_ref, bg_ref, w1_ref, b1_ref, w2_ref, b2_ref, w3_ref, b3_ref,
         out_ref) = refs[NBk:]
        dmerged = dp_ref[0] + dp_ref[1]
        dinv = 1.0 / (dmerged[:, 0:1] + 1e-16)
        h1 = None
        for k in range(NBk):
            Xk = jnp.maximum((p_refs[k][0] + p_refs[k][1]) * dinv
                             + bg_ref[k:k + 1, :], 0.0)
            contrib = _dot(Xk, w1_ref[k * OW_in:(k + 1) * OW_in, :])
            h1 = contrib if h1 is None else h1 + contrib
        h1 = jnp.maximum(h1 + b1_ref[0:1, :], 0.0)
        h2 = jnp.maximum(_dot(h1, w2_ref[...]) + b2_ref[0:1, :], 0.0)
        out_ref[...] = jnp.maximum(_dot(h2, w3_ref[...]) + b3_ref[0:1, :], 0.0)

    in_specs = [pl.BlockSpec((2, 200, OW_in), lambda nb: (0, nb, 0))
                for _ in range(NBk)]
    in_specs += [pl.BlockSpec((2, 200, 8), lambda nb: (0, nb, 0)),
                 pl.BlockSpec((NBk, OW_in), lambda nb: (0, 0)),
                 pl.BlockSpec((128, 256), lambda nb: (0, 0)),
                 pl.BlockSpec((1, 256), lambda nb: (0, 0)),
                 pl.BlockSpec((256, 128), lambda nb: (0, 0)),
                 pl.BlockSpec((1, 128), lambda nb: (0, 0)),
                 pl.BlockSpec((128, 128), lambda nb: (0, 0)),
                 pl.BlockSpec((1, 128), lambda nb: (0, 0))]
    return pl.pallas_call(
        body,
        grid=(NBLK,),
        in_specs=in_specs,
        out_specs=pl.BlockSpec((200, 128), lambda nb: (nb, 0)),
        out_shape=jax.ShapeDtypeStruct((N, 128), F32),
    )(*p_list, dp2, bprev, W1, b1.reshape(1, 256),
      W2, b2.reshape(1, 128), W3, b3.reshape(1, 128))


def _final_matmul(xf, yf):
    def body(x_ref, y_ref, out_ref):
        out_ref[...] = lax.dot_general(
            x_ref[...], y_ref[...], (((1,), (1,)), ((), ())),
            precision=_HIGH, preferred_element_type=F32)

    return pl.pallas_call(
        body,
        grid=(M // 400,),
        in_specs=[pl.BlockSpec((400, 128), lambda i: (i, 0)),
                  pl.BlockSpec((D, 128), lambda i: (0, 0))],
        out_specs=pl.BlockSpec((400, D), lambda i: (i, 0)),
        out_shape=jax.ShapeDtypeStruct((M, D), F32),
    )(xf, yf)


def _gat_edges(ei, xw_list, atabs, c16, N, E, H, C, OW):
    src_arr = ei[0]
    dst_arr = ei[1]
    NB = len(xw_list)
    GRP = OW // 16
    EPT = E // NW
    NCHUNK = EPT // CH
    RPT = 8 * -(-N // (NS * 8))
    N_pad = NS * RPT
    DPT = (N * 8) // NS
    ZB = 1280
    mesh = plsc.VectorSubcoreMesh(core_axis_name="c", subcore_axis_name="s")

    out_type = [jax.ShapeDtypeStruct((2, N_pad, OW), F32) for _ in range(NB)]
    out_type.append(jax.ShapeDtypeStruct((N * 8,), F32))
    out_type.append(jax.ShapeDtypeStruct((N * 8,), F32))

    scratch = [
        pltpu.VMEM((N * 2,), F32),
        pltpu.VMEM((16,), F32),
        pltpu.VMEM((CH,), I32),
        pltpu.VMEM((CH,), I32),
        pltpu.VMEM((CH,), F32),
        pltpu.VMEM((CH,), I32),
        pltpu.VMEM((CH, OW), F32),
        pltpu.VMEM((ZB,), F32),
        pltpu.SemaphoreType.DMA,
        pltpu.VMEM_SHARED((N_pad, OW), F32),
        pltpu.VMEM_SHARED((N * 8,), F32),
    ]

    @functools.partial(pl.kernel, out_type=out_type, mesh=mesh,
                       compiler_params=pltpu.CompilerParams(
                           needs_layout_passes=False),
                       scratch_types=scratch)
    def kfn(src_h, dst_h, *rest):
        xw_h = rest[:NB]
        atab_h = rest[NB:NB + H]
        c_h = rest[NB + H]
        p_h = rest[NB + H + 1:NB + H + 1 + NB]
        dp0_h = rest[NB + H + 1 + NB]
        dp1_h = rest[NB + H + 2 + NB]
        (atab_v, c_v, src_v, dst_v, tpass_v, didx_v,
         rowbuf, zb_v, sem, acc_sh, dacc_sh) = rest[NB + H + 3 + NB:]

        sc = lax.axis_index("c")
        sub = lax.axis_index("s")
        wid = sub * NC + sc
        ebase = wid * EPT

        pltpu.sync_copy(c_h, c_v)
        cvec = c_v[...]

        zeros16 = jnp.zeros((16,), F32)

        def zb_body(i, _):
            zb_v[pl.ds(i * 16, 16)] = zeros16
            return 0
        lax.fori_loop(0, ZB // 16, zb_body, 0)

        def zero_rowbuf():
            def rb_body(i, _):
                for g in range(GRP):
                    rowbuf[i, pl.ds(g * 16, 16)] = zeros16
                return 0
            lax.fori_loop(0, CH, rb_body, 0)

        def zero_shared_rows(dst_ref, base_row, nrows):
            zero_rowbuf()
            off = 0
            while off < nrows:
                sz = min(CH, nrows - off)
                pltpu.sync_copy(rowbuf.at[pl.ds(0, sz)],
                                dst_ref.at[pl.ds(base_row + off, sz)])
                off += sz

        def zero_shared_flat(dst_ref, base, n):
            off = 0
            while off < n:
                sz = min(ZB, n - off)
                pltpu.sync_copy(zb_v.at[pl.ds(0, sz)],
                                dst_ref.at[pl.ds(base + off, sz)])
                off += sz

        idx16 = jnp.zeros((16,), I32)

        def compute_t(s16, d16):
            sa = plsc.load_gather(atab_v, [s16 * 2])
            da = plsc.load_gather(atab_v, [d16 * 2 + 1])
            e = sa + da
            e = jnp.maximum(e, 0.2 * e)
            return jnp.exp(e - cvec)

        for j in range(NB):
            hj = (j * OW) // C
            first_of_head = (j * OW) % C == 0
            if first_of_head:
                pltpu.sync_copy(atab_h[hj], atab_v)
            zero_shared_rows(acc_sh, sub * RPT, RPT)
            if j == 0:
                zero_shared_flat(dacc_sh, sub * DPT, DPT)
            plsc.subcore_barrier()

            def chunk_body(ci, _):
                e0 = ebase + ci * CH
                pltpu.sync_copy(src_h.at[pl.ds(e0, CH)], src_v)
                pltpu.sync_copy(dst_h.at[pl.ds(e0, CH)], dst_v)

                def grp_body(g, _):
                    s16 = src_v[pl.ds(g * 16, 16)]
                    d16 = dst_v[pl.ds(g * 16, 16)]
                    tp = compute_t(s16, d16)
                    tpass_v[pl.ds(g * 16, 16)] = tp
                    if first_of_head:
                        didx_v[pl.ds(g * 16, 16)] = d16 * 8 + hj
                    return 0
                lax.fori_loop(0, CH // 16, grp_body, 0)

                if first_of_head:
                    pltpu.sync_copy(tpass_v, dacc_sh.at[didx_v], add=True)

                pltpu.async_copy(xw_h[j].at[src_v], rowbuf, sem).wait()

                def row_body(i, _):
                    ts = plsc.load_gather(tpass_v, [idx16 + i])
                    for g in range(GRP):
                        v = rowbuf[i, pl.ds(g * 16, 16)]
                        rowbuf[i, pl.ds(g * 16, 16)] = v * ts
                    return 0
                lax.fori_loop(0, CH, row_body, 0)

                pltpu.sync_copy(rowbuf, acc_sh.at[dst_v], add=True)
                return 0
            lax.fori_loop(0, NCHUNK, chunk_body, 0)

            plsc.subcore_barrier()
            off = 0
            while off < RPT:
                sz = min(CH, RPT - off)
                pltpu.sync_copy(acc_sh.at[pl.ds(sub * RPT + off, sz)],
                                rowbuf.at[pl.ds(0, sz)])
                pltpu.sync_copy(rowbuf.at[pl.ds(0, sz)],
                                p_h[j].at[sc, pl.ds(sub * RPT + off, sz)])
                off += sz
            if j == NB - 1:
                off = 0
                while off < DPT:
                    sz = min(ZB, DPT - off)
                    pltpu.sync_copy(dacc_sh.at[pl.ds(sub * DPT + off, sz)],
                                    zb_v.at[pl.ds(0, sz)])

                    @pl.when(sc == 0)
                    def _():
                        pltpu.sync_copy(zb_v.at[pl.ds(0, sz)],
                                        dp0_h.at[pl.ds(sub * DPT + off, sz)])

                    @pl.when(sc == 1)
                    def _():
                        pltpu.sync_copy(zb_v.at[pl.ds(0, sz)],
                                        dp1_h.at[pl.ds(sub * DPT + off, sz)])
                    off += sz
            plsc.subcore_barrier()

    atabs_flat = [t.reshape(-1) for t in atabs]
    outs = kfn(src_arr, dst_arr, *xw_list, *atabs_flat, c16)
    dp = jnp.stack([outs[NB], outs[NB + 1]], axis=0)
    return list(outs[:NB]), dp


def kernel(x_m, x_d, mm_edge_index, dd_edge_index,
           gx1_W, gx1_as, gx1_ad, gx1_b, gx2_W, gx2_as, gx2_ad, gx2_b,
           gy1_W, gy1_as, gy1_ad, gy1_b, gy2_W, gy2_as, gy2_ad, gy2_b,
           lx1_W, lx1_b, lx2_W, lx2_b, lx3_W, lx3_b,
           ly1_W, ly1_b, ly2_W, ly2_b, ly3_W, ly3_b):
    def graph_chain(x0, ei, N, E, OW, g1_W, g1_as, g1_ad, g1_b,
                    g2_W, g2_as, g2_ad, g2_b, m1_W, m1_b, m2_W, m2_b,
                    m3_W, m3_b):
        Fin = x0.shape[1]
        xw1, atab1, c1 = _proj_raw(x0, g1_W, g1_as, g1_ad, N, Fin,
                                   2, 2 * Fin, OW)
        p1, dp1 = _gat_edges(ei, xw1, atab1, c1, N, E, 2, 2 * Fin, OW)
        xw2, atab2, c2 = _proj_merge(p1, dp1, g1_b, g2_W, g2_as, g2_ad,
                                     N, 2 * Fin, 1, Fin, OW, OW)
        p2, dp2 = _gat_edges(ei, xw2, atab2, c2, N, E, 1, Fin, OW)
        return _merge_mlp(p2, dp2, g2_b, m1_W, m1_b, m2_W, m2_b,
                          m3_W, m3_b, N, OW)

    xf = graph_chain(x_m, mm_edge_index, M, 320000, 128,
                     gx1_W, gx1_as, gx1_ad, gx1_b,
                     gx2_W, gx2_as, gx2_ad, gx2_b,
                     lx1_W, lx1_b, lx2_W, lx2_b, lx3_W, lx3_b)
    yf = graph_chain(x_d, dd_edge_index, D, 64000, 128,
                     gy1_W, gy1_as, gy1_ad, gy1_b,
                     gy2_W, gy2_as, gy2_ad, gy2_b,
                     ly1_W, ly1_b, ly2_W, ly2_b, ly3_W, ly3_b)
    return _final_matmul(xf, yf)
